# baseline (speedup 1.0000x reference)
import jax
import jax.numpy as jnp
from jax.experimental import pallas as pl
from jax.experimental.pallas import tpu as pltpu

B = 8192
FEAT = 1024
HID = 2048
TILE_B = 256


def _fused_kernel(x0_ref, x1_ref, x2_ref,
                  W1_0_ref, b1_0_ref, W2_0_ref, b2_0_ref,
                  W1_1_ref, b1_1_ref, W2_1_ref, b2_1_ref,
                  W1_2_ref, b1_2_ref, W2_2_ref, b2_2_ref,
                  Wg_ref, bg_ref,
                  fused_ref, gw_ref):
    xs = (x0_ref[...], x1_ref[...], x2_ref[...])

    logits = bg_ref[...]
    for i, x in enumerate(xs):
        logits = logits + jnp.dot(x, Wg_ref[i * FEAT:(i + 1) * FEAT, :],
                                  preferred_element_type=jnp.float32)

    l0 = logits[:, 0:1]
    l1 = logits[:, 1:2]
    l2 = logits[:, 2:3]
    r0 = (l1 > l0).astype(jnp.float32) + (l2 > l0).astype(jnp.float32)
    r1 = (l0 >= l1).astype(jnp.float32) + (l2 > l1).astype(jnp.float32)
    r2 = (l0 >= l2).astype(jnp.float32) + (l1 >= l2).astype(jnp.float32)
    m0 = jnp.where(r0 < 2.0, l0, 0.0)
    m1 = jnp.where(r1 < 2.0, l1, 0.0)
    m2 = jnp.where(r2 < 2.0, l2, 0.0)
    mx = jnp.maximum(jnp.maximum(m0, m1), m2)
    e0 = jnp.exp(m0 - mx)
    e1 = jnp.exp(m1 - mx)
    e2 = jnp.exp(m2 - mx)
    denom = e0 + e1 + e2
    w0 = e0 / denom
    w1 = e1 / denom
    w2 = e2 / denom
    gw_ref[...] = jnp.concatenate([w0, w1, w2], axis=1)

    params = ((W1_0_ref, b1_0_ref, W2_0_ref, b2_0_ref),
              (W1_1_ref, b1_1_ref, W2_1_ref, b2_1_ref),
              (W1_2_ref, b1_2_ref, W2_2_ref, b2_2_ref))
    ws = (w0, w1, w2)
    acc = None
    for x, (W1_ref, b1_ref, W2_ref, b2_ref), w in zip(xs, params, ws):
        h = jnp.maximum(
            jnp.dot(x, W1_ref[...].astype(jnp.float32),
                    preferred_element_type=jnp.float32)
            + b1_ref[...], 0.0)
        y = jnp.dot(h, W2_ref[...].astype(jnp.float32),
                    preferred_element_type=jnp.float32) + b2_ref[...]
        acc = w * y if acc is None else acc + w * y
    fused_ref[...] = acc


@jax.jit
def kernel(doc_word, doc_pos, doc_entity,
           W1_0, b1_0, W2_0, b2_0,
           W1_1, b1_1, W2_1, b2_1,
           W1_2, b1_2, W2_2, b2_2,
           Wg, bg):
    grid = (B // TILE_B,)

    row_spec = pl.BlockSpec((TILE_B, FEAT), lambda r: (r, 0))
    full = lambda shape: pl.BlockSpec(shape, lambda r: (0,) * len(shape))

    w1_spec = full((FEAT, HID))
    b1_spec = full((1, HID))
    w2_spec = full((HID, FEAT))
    b2_spec = full((1, FEAT))

    in_specs = [row_spec, row_spec, row_spec]
    for _ in range(3):
        in_specs += [w1_spec, b1_spec, w2_spec, b2_spec]
    in_specs += [full((3 * FEAT, 3)), full((1, 3))]

    out_specs = (pl.BlockSpec((TILE_B, FEAT), lambda r: (r, 0)),
                 pl.BlockSpec((TILE_B, 3), lambda r: (r, 0)))

    fused, gw = pl.pallas_call(
        _fused_kernel,
        grid=grid,
        in_specs=in_specs,
        out_specs=out_specs,
        out_shape=(jax.ShapeDtypeStruct((B, FEAT), jnp.float32),
                   jax.ShapeDtypeStruct((B, 3), jnp.float32)),
        compiler_params=pltpu.CompilerParams(
            dimension_semantics=("arbitrary",),
            vmem_limit_bytes=64 * 1024 * 1024,
        ),
    )(doc_word, doc_pos, doc_entity,
      W1_0.astype(jnp.bfloat16), b1_0.reshape(1, HID),
      W2_0.astype(jnp.bfloat16), b2_0.reshape(1, FEAT),
      W1_1.astype(jnp.bfloat16), b1_1.reshape(1, HID),
      W2_1.astype(jnp.bfloat16), b2_1.reshape(1, FEAT),
      W1_2.astype(jnp.bfloat16), b1_2.reshape(1, HID),
      W2_2.astype(jnp.bfloat16), b2_2.reshape(1, FEAT),
      Wg, bg.reshape(1, 3))
    return (fused, gw)


# speedup vs baseline: 1.0648x; 1.0648x over previous
import jax
import jax.numpy as jnp
from jax.experimental import pallas as pl
from jax.experimental.pallas import tpu as pltpu

B = 8192
FEAT = 1024
HID = 2048
TILE_B = 256


def _fused_kernel(x0_ref, x1_ref, x2_ref,
                  W1_0_ref, b1_0_ref, W2_0_ref, b2_0_ref,
                  W1_1_ref, b1_1_ref, W2_1_ref, b2_1_ref,
                  W1_2_ref, b1_2_ref, W2_2_ref, b2_2_ref,
                  Wg_ref, bg_ref,
                  fused_ref, gw_ref):
    xs = (x0_ref[...], x1_ref[...], x2_ref[...])

    logits = bg_ref[...]
    for i, x in enumerate(xs):
        logits = logits + jnp.dot(x, Wg_ref[i * FEAT:(i + 1) * FEAT, :],
                                  preferred_element_type=jnp.float32)

    l0 = logits[:, 0:1]
    l1 = logits[:, 1:2]
    l2 = logits[:, 2:3]
    r0 = (l1 > l0).astype(jnp.float32) + (l2 > l0).astype(jnp.float32)
    r1 = (l0 >= l1).astype(jnp.float32) + (l2 > l1).astype(jnp.float32)
    r2 = (l0 >= l2).astype(jnp.float32) + (l1 >= l2).astype(jnp.float32)
    m0 = jnp.where(r0 < 2.0, l0, 0.0)
    m1 = jnp.where(r1 < 2.0, l1, 0.0)
    m2 = jnp.where(r2 < 2.0, l2, 0.0)
    mx = jnp.maximum(jnp.maximum(m0, m1), m2)
    e0 = jnp.exp(m0 - mx)
    e1 = jnp.exp(m1 - mx)
    e2 = jnp.exp(m2 - mx)
    denom = e0 + e1 + e2
    w0 = e0 / denom
    w1 = e1 / denom
    w2 = e2 / denom
    gw_ref[...] = jnp.concatenate([w0, w1, w2], axis=1)

    params = ((W1_0_ref, b1_0_ref, W2_0_ref, b2_0_ref),
              (W1_1_ref, b1_1_ref, W2_1_ref, b2_1_ref),
              (W1_2_ref, b1_2_ref, W2_2_ref, b2_2_ref))
    ws = (w0, w1, w2)
    acc = None
    for x, (W1_ref, b1_ref, W2_ref, b2_ref), w in zip(xs, params, ws):
        h = jnp.maximum(
            jnp.dot(x, W1_ref[...], preferred_element_type=jnp.float32)
            + b1_ref[...], 0.0)
        y = jnp.dot(h, W2_ref[...], preferred_element_type=jnp.float32) \
            + b2_ref[...]
        acc = w * y if acc is None else acc + w * y
    fused_ref[...] = acc


@jax.jit
def kernel(doc_word, doc_pos, doc_entity,
           W1_0, b1_0, W2_0, b2_0,
           W1_1, b1_1, W2_1, b2_1,
           W1_2, b1_2, W2_2, b2_2,
           Wg, bg):
    grid = (B // TILE_B,)

    row_spec = pl.BlockSpec((TILE_B, FEAT), lambda r: (r, 0))
    full = lambda shape: pl.BlockSpec(shape, lambda r: (0,) * len(shape))

    w1_spec = full((FEAT, HID))
    b1_spec = full((1, HID))
    w2_spec = full((HID, FEAT))
    b2_spec = full((1, FEAT))

    in_specs = [row_spec, row_spec, row_spec]
    for _ in range(3):
        in_specs += [w1_spec, b1_spec, w2_spec, b2_spec]
    in_specs += [full((3 * FEAT, 3)), full((1, 3))]

    out_specs = (pl.BlockSpec((TILE_B, FEAT), lambda r: (r, 0)),
                 pl.BlockSpec((TILE_B, 3), lambda r: (r, 0)))

    fused, gw = pl.pallas_call(
        _fused_kernel,
        grid=grid,
        in_specs=in_specs,
        out_specs=out_specs,
        out_shape=(jax.ShapeDtypeStruct((B, FEAT), jnp.float32),
                   jax.ShapeDtypeStruct((B, 3), jnp.float32)),
        compiler_params=pltpu.CompilerParams(
            dimension_semantics=("parallel",),
            vmem_limit_bytes=64 * 1024 * 1024,
        ),
    )(doc_word, doc_pos, doc_entity,
      W1_0, b1_0.reshape(1, HID), W2_0, b2_0.reshape(1, FEAT),
      W1_1, b1_1.reshape(1, HID), W2_1, b2_1.reshape(1, FEAT),
      W1_2, b1_2.reshape(1, HID), W2_2, b2_2.reshape(1, FEAT),
      Wg, bg.reshape(1, 3))
    return (fused, gw)
